# Initial kernel scaffold; baseline (speedup 1.0000x reference)
import jax
import jax.numpy as jnp
from jax.experimental import pallas as pl


def kernel(t, x, edge_index, edge_weight, x0, alpha_train, w, d):
    raise NotImplementedError("write your pallas kernel here")


# speedup vs baseline: 4.1027x; 4.1027x over previous
import functools

import jax
import jax.numpy as jnp
from jax import lax
from jax.experimental import pallas as pl
from jax.experimental.pallas import tpu as pltpu
from jax.experimental.pallas import tpu_sc as plsc

N = 10000
E = 320000
D = 128

NC = 2
NS = 16
CH = 80
EC = E // NC
ET = EC // NS
NCHUNK = ET // CH
NPAD = 10240
RT = NPAD // NS
ZR = 80


def _make_sc_spmm():
    mesh = plsc.VectorSubcoreMesh(core_axis_name="c", subcore_axis_name="s",
                                  num_cores=NC, num_subcores=NS)

    @functools.partial(
        pl.kernel,
        out_type=jax.ShapeDtypeStruct((NC, N, D), jnp.float32),
        mesh=mesh,
        scratch_types=[
            pltpu.VMEM((CH,), jnp.int32),
            pltpu.VMEM((CH,), jnp.int32),
            pltpu.VMEM((CH,), jnp.float32),
            pltpu.VMEM((CH, D), jnp.float32),
            pltpu.VMEM((ZR, D), jnp.float32),
            pltpu.VMEM_SHARED((NPAD, D), jnp.float32),
            pltpu.SemaphoreType.DMA,
        ],
    )
    def sc_spmm(x_hbm, col_hbm, row_hbm, ew_hbm, out_hbm,
                colv, rowv, wv, gv, zv, acc, sem):
        c = lax.axis_index("c")
        s = lax.axis_index("s")

        def zfill(r, _):
            for j in range(D // 16):
                zv[r, pl.ds(j * 16, 16)] = jnp.zeros((16,), jnp.float32)
            return 0
        lax.fori_loop(0, ZR, zfill, 0)
        r0 = s * RT
        nch = (jnp.minimum(N - r0, RT)) // ZR

        def zcopy(k, _):
            pltpu.sync_copy(zv, acc.at[pl.ds(r0 + k * ZR, ZR)])
            return 0
        lax.fori_loop(0, nch, zcopy, 0)
        plsc.subcore_barrier()

        tile_base = c * EC + s * ET

        def chunk_body(ch, _):
            base = tile_base + ch * CH
            pltpu.sync_copy(col_hbm.at[pl.ds(base, CH)], colv)
            pltpu.sync_copy(row_hbm.at[pl.ds(base, CH)], rowv)
            pltpu.sync_copy(ew_hbm.at[pl.ds(base, CH)], wv)
            pltpu.async_copy(x_hbm.at[colv], gv, sem).wait()

            def scale(g, _):
                wvec = wv[pl.ds(g * 16, 16)]
                for l in range(16):
                    we = wvec[l]
                    e = g * 16 + l
                    for j in range(D // 16):
                        sl = pl.ds(j * 16, 16)
                        gv[e, sl] = gv[e, sl] * we
                return 0
            lax.fori_loop(0, CH // 16, scale, 0)

            pltpu.sync_copy(gv, acc.at[rowv], add=True)
            return 0

        lax.fori_loop(0, NCHUNK, chunk_body, 0)
        plsc.subcore_barrier()

        def wcopy(k, _):
            rr = r0 + k * ZR
            pltpu.sync_copy(acc.at[pl.ds(rr, ZR)], out_hbm.at[c, pl.ds(rr, ZR)])
            return 0
        lax.fori_loop(0, nch, wcopy, 0)

    return sc_spmm


_sc_spmm_cache = []


def _sc_spmm(x, col, row, ew):
    if not _sc_spmm_cache:
        _sc_spmm_cache.append(_make_sc_spmm())
    return _sc_spmm_cache[0](x, col, row, ew)


def _combine_body(x_ref, ax0_ref, ax1_ref, x0_ref, al_ref, w_ref, d_ref,
                  o_ref, wm_ref):
    @pl.when(pl.program_id(0) == 0)
    def _():
        dc = jnp.clip(d_ref[...], 0.0, 1.0)
        wd = w_ref[...] * dc
        wm = lax.dot_general(wd, w_ref[...], (((1,), (1,)), ((), ())),
                             preferred_element_type=jnp.float32)
        ii = lax.broadcasted_iota(jnp.int32, (D, D), 0)
        jj = lax.broadcasted_iota(jnp.int32, (D, D), 1)
        eye = jnp.where(ii == jj, 1.0, 0.0).astype(jnp.float32)
        wm_ref[...] = wm - eye

    a = jax.nn.sigmoid(al_ref[...])
    xb = x_ref[...]
    axs = ax0_ref[...] + ax1_ref[...]
    xw = jnp.dot(xb, wm_ref[...], preferred_element_type=jnp.float32)
    o_ref[...] = 0.5 * a * (axs - xb) + xw + x0_ref[...]


def _combine(x, ax0, ax1, x0, alpha2d, w, d2d):
    BN = 1000
    grid = (N // BN,)
    row_spec = pl.BlockSpec((BN, D), lambda i: (i, 0))
    return pl.pallas_call(
        _combine_body,
        grid=grid,
        in_specs=[
            row_spec, row_spec, row_spec, row_spec,
            pl.BlockSpec((BN, 1), lambda i: (i, 0)),
            pl.BlockSpec((D, D), lambda i: (0, 0)),
            pl.BlockSpec((1, D), lambda i: (0, 0)),
        ],
        out_specs=row_spec,
        out_shape=jax.ShapeDtypeStruct((N, D), jnp.float32),
        scratch_shapes=[pltpu.VMEM((D, D), jnp.float32)],
    )(x, ax0, ax1, x0, alpha2d, w, d2d)


def kernel(t, x, edge_index, edge_weight, x0, alpha_train, w, d):
    ei = edge_index.astype(jnp.int32)
    row = ei[0]
    col = ei[1]
    axp = _sc_spmm(x, col, row, edge_weight)
    f = _combine(x, axp[0], axp[1], x0, alpha_train.reshape(N, 1), w,
                 d.reshape(1, D))
    return f
